# baseline (speedup 1.0000x reference)
import functools

import jax
import jax.numpy as jnp
from jax import lax
from jax.experimental import pallas as pl
from jax.experimental.pallas import tpu as pltpu
from jax.experimental.pallas import tpu_sc as plsc

_NC = 2
_NS = 16
_NW = _NC * _NS
_L = 16
_C = 16
_NBUF = 3


def _sc_modality_norm(feat, gamma, beta, mid16):
    n, d = feat.shape
    nm = gamma.shape[0]
    rows_per_w = n // _NW
    nchunks = rows_per_w // _C
    ngroups = d // _L

    mesh = plsc.VectorSubcoreMesh(
        core_axis_name="c", subcore_axis_name="s", num_cores=_NC, num_subcores=_NS
    )

    @functools.partial(
        pl.kernel,
        out_type=jax.ShapeDtypeStruct((n, d), jnp.float32),
        mesh=mesh,
        scratch_types=[
            pltpu.VMEM((nm, d), jnp.float32),
            pltpu.VMEM((nm, d), jnp.float32),
            pltpu.VMEM((_L,), jnp.int32),
            pltpu.VMEM((d,), jnp.float32),
            pltpu.VMEM((d,), jnp.float32),
            [pltpu.VMEM((_C, d), jnp.float32) for _ in range(_NBUF)],
            [pltpu.SemaphoreType.DMA for _ in range(_NBUF)],
            [pltpu.SemaphoreType.DMA for _ in range(_NBUF)],
        ],
    )
    def run(feat_hbm, gamma_hbm, beta_hbm, mid_hbm, out_hbm,
            g_v, b_v, mid_v, gsel_v, bsel_v, bufs, lsems, ssems):
        wid = lax.axis_index("s") * _NC + lax.axis_index("c")
        base = wid * rows_per_w

        def start_load(k):
            bi = k % _NBUF
            return pltpu.async_copy(
                feat_hbm.at[pl.ds(base + k * _C, _C)], bufs[bi], lsems[bi])

        loads = {0: start_load(0), 1: start_load(1)}

        pltpu.sync_copy(gamma_hbm, g_v)
        pltpu.sync_copy(beta_hbm, b_v)
        pltpu.sync_copy(mid_hbm, mid_v)
        midv = mid_v[...]

        def sel_body(j, _):
            sl = pl.ds(j * _L, _L)
            g = g_v[0, sl]
            b = b_v[0, sl]
            for m in range(1, nm):
                pick = midv == m
                g = jnp.where(pick, g_v[m, sl], g)
                b = jnp.where(pick, b_v[m, sl], b)
            gsel_v[sl] = g
            bsel_v[sl] = b
            return 0

        lax.fori_loop(0, ngroups, sel_body, 0)

        def compute(buf):
            @plsc.parallel_loop(0, d, step=_L)
            def col_body(c):
                sl = pl.ds(c, _L)
                g = gsel_v[sl]
                b = bsel_v[sl]
                for r in range(_C):
                    buf[r, sl] = buf[r, sl] * g + b

        stores = {}
        waited = set()
        for k in range(nchunks):
            bi = k % _NBUF
            loads[k].wait()
            compute(bufs[bi])
            plsc.subcore_barrier()
            stores[k] = pltpu.async_copy(
                bufs[bi], out_hbm.at[pl.ds(base + k * _C, _C)], ssems[bi])
            if k + 2 < nchunks:
                if k >= 1:
                    stores[k - 1].wait()
                    waited.add(k - 1)
                loads[k + 2] = start_load(k + 2)
        for k in range(nchunks):
            if k not in waited:
                stores[k].wait()

    return run(feat, gamma, beta, mid16)


def kernel(feat, gamma, beta, modality_id):
    nm = gamma.shape[0]
    mid = jnp.clip(jnp.asarray(modality_id, dtype=jnp.int32), 0, nm - 1)
    mid16 = jnp.full((_L,), mid, dtype=jnp.int32)
    return _sc_modality_norm(feat, gamma, beta, mid16)
